# baseline (speedup 1.0000x reference)
import dataclasses
import functools

import jax
import jax.numpy as jnp
from jax import lax
from jax.experimental import pallas as pl
from jax.experimental.pallas import tpu as pltpu
from jax.experimental.pallas import tpu_sc as plsc

N = 10000
E = 320000
D_IN = 128
HID = 64
OUT = 32
EMB = 128
HEADS = 2

NC = 2
NS = 16
NW = NC * NS
EPW = E // NW
CH = 80
NCHUNK = EPW // CH
LL = 16


def _h0_body(x_ref, n2v_ref, Wn_ref, bn_ref, Win_ref, bin_ref, Wg_ref, bg_ref,
             o_ref):
    x = x_ref[...]
    emb_p = n2v_ref[...] @ Wn_ref[...].T + bn_ref[...]
    Win = Win_ref[...]
    raw = x @ Win[:, :D_IN].T + bin_ref[...]
    xp = raw + emb_p @ Win[:, D_IN:].T
    Wg = Wg_ref[...]
    g = jax.nn.sigmoid(x @ Wg[:, :D_IN].T + emb_p @ Wg[:, D_IN:].T + bg_ref[...])
    o_ref[...] = g * xp + (1.0 - g) * raw


def _h0(x, n2v_table, W_n2vp, b_n2vp, W_in, b_in, W_gate, b_gate):
    blk = 2000
    grid = (N // blk,)
    row_spec = pl.BlockSpec((blk, D_IN), lambda i: (i, 0))
    full = lambda shape: pl.BlockSpec(shape, lambda i: (0,) * len(shape))
    return pl.pallas_call(
        _h0_body,
        grid=grid,
        in_specs=[row_spec, row_spec, full((HID, EMB)), full((1, HID)),
                  full((HID, D_IN + HID)), full((1, HID)),
                  full((HID, D_IN + HID)), full((1, HID))],
        out_specs=pl.BlockSpec((blk, HID), lambda i: (i, 0)),
        out_shape=jax.ShapeDtypeStruct((N, HID), jnp.float32),
    )(x, n2v_table, W_n2vp, b_n2vp.reshape(1, HID), W_in,
      b_in.reshape(1, HID), W_gate, b_gate.reshape(1, HID))


def _stripe_copy(src_ref, dst_ref, sid, nrows):
    rp = (nrows // NS) & ~7
    tail = nrows - NS * rp
    pltpu.sync_copy(src_ref.at[pl.ds(sid * rp, rp)],
                    dst_ref.at[pl.ds(sid * rp, rp)])
    if tail:
        @pl.when(sid == 0)
        def _():
            pltpu.sync_copy(src_ref.at[pl.ds(NS * rp, tail)],
                            dst_ref.at[pl.ds(NS * rp, tail)])


def _edge_ring(h_hbm, gidx2, sidx2, acc_sh, rows_a, rows_b, sem_a, sem_b):
    pltpu.async_copy(h_hbm.at[gidx2.at[0]], rows_a, sem_a)

    @pl.loop(0, NCHUNK - 1, step=2)
    def _(i):
        pltpu.make_async_copy(h_hbm.at[gidx2.at[i]], rows_a, sem_a).wait()
        pltpu.async_copy(h_hbm.at[gidx2.at[i + 1]], rows_b, sem_b)
        pltpu.sync_copy(rows_a, acc_sh.at[sidx2.at[i]], add=True)
        pltpu.make_async_copy(h_hbm.at[gidx2.at[i + 1]], rows_b, sem_b).wait()
        pltpu.async_copy(h_hbm.at[gidx2.at[i + 2]], rows_a, sem_a)
        pltpu.sync_copy(rows_b, acc_sh.at[sidx2.at[i + 1]], add=True)

    last = NCHUNK - 1
    pltpu.make_async_copy(h_hbm.at[gidx2.at[last]], rows_a, sem_a).wait()
    pltpu.sync_copy(rows_a, acc_sh.at[sidx2.at[last]], add=True)


D80 = 80
MP = N + 8
_MESH = plsc.VectorSubcoreMesh(core_axis_name="c", subcore_axis_name="s")
_CP = pltpu.CompilerParams(use_tc_tiling_on_sc=False)
if "needs_layout_passes" in pltpu.CompilerParams.__dataclass_fields__:
    _CP = dataclasses.replace(_CP, needs_layout_passes=False)


def _edge_kernel_body(t_hbm, als_hbm, ald_hbm, flag_hbm, src_hbm, dst_hbm,
                      z_hbm, out_hbm, src2_v, dst2_v, gidx2_v, sidx2_v,
                      rows_a, rows_b, als_v, ald_v, flag_v, acc_sh, sem_a,
                      sem_b):
    cid = lax.axis_index("c")
    sid = lax.axis_index("s")
    wid = sid * NC + cid
    pltpu.sync_copy(src_hbm.at[wid], src2_v)
    pltpu.sync_copy(dst_hbm.at[wid], dst2_v)
    pltpu.sync_copy(als_hbm, als_v)
    pltpu.sync_copy(ald_hbm, ald_v)
    pltpu.sync_copy(flag_hbm, flag_v)
    _stripe_copy(z_hbm, acc_sh, sid, MP)

    flip = flag_v[...] > 0.5
    dump = jnp.full((LL,), N, jnp.int32)

    @pl.loop(0, NCHUNK)
    def _(i):
        for kk in range(CH // LL):
            s16 = src2_v[i, pl.ds(kk * LL, LL)]
            d16 = dst2_v[i, pl.ds(kk * LL, LL)]
            a = plsc.load_gather(als_v, [s16])
            b = plsc.load_gather(ald_v, [d16])
            keep = jnp.logical_xor((a + b) > 0.0, flip)
            gidx2_v[i, pl.ds(kk * LL, LL)] = jnp.where(keep, s16, dump)
            sidx2_v[i, pl.ds(kk * LL, LL)] = jnp.where(keep, d16, dump)

    plsc.subcore_barrier()
    _edge_ring(t_hbm, gidx2_v, sidx2_v, acc_sh, rows_a, rows_b, sem_a, sem_b)
    plsc.subcore_barrier()
    _stripe_copy(acc_sh, out_hbm.at[cid], sid, MP)


_EDGE_KERNEL = pl.kernel(
    _edge_kernel_body,
    out_type=jax.ShapeDtypeStruct((NC, MP, D80), jnp.float32),
    mesh=_MESH,
    compiler_params=_CP,
    scratch_types=[
        pltpu.VMEM((NCHUNK, CH), jnp.int32),
        pltpu.VMEM((NCHUNK, CH), jnp.int32),
        pltpu.VMEM((NCHUNK, CH), jnp.int32),
        pltpu.VMEM((NCHUNK, CH), jnp.int32),
        pltpu.VMEM((CH, D80), jnp.float32),
        pltpu.VMEM((CH, D80), jnp.float32),
        pltpu.VMEM((N,), jnp.float32),
        pltpu.VMEM((N,), jnp.float32),
        pltpu.VMEM((LL,), jnp.float32),
        pltpu.VMEM_SHARED((MP, D80), jnp.float32),
        pltpu.SemaphoreType.DMA,
        pltpu.SemaphoreType.DMA,
    ],
)

_DUMP = None


def _edge_pass(T, als, ald, flag, src3, dst3):
    Tp = jnp.concatenate([T, jnp.zeros((8, D80), jnp.float32)], axis=0)
    zeros = jnp.zeros((MP, D80), jnp.float32)
    flagv = jnp.full((LL,), flag, jnp.float32)
    p = _EDGE_KERNEL(Tp, als, ald, flagv, src3, dst3, zeros)
    return p[0, :N] + p[1, :N]


_ONES = None


def _segsum(h_ext, src3, dst3):
    ones = jnp.ones((N,), jnp.float32)
    return _edge_pass(h_ext, ones, ones, 0.0, src3, dst3)


def _gat_head(xw_h, als, ald, src3, dst3, token):
    n = xw_h.shape[0]
    asm = jnp.max(als)
    adm = jnp.max(ald)
    bb = asm + adm
    mb = jnp.maximum(bb, 0.2 * bb)
    fpos = jnp.exp(als - asm)
    fneg = jnp.exp(0.2 * (als - asm))
    pad = jnp.zeros((n, 15), jnp.float32)
    Tp = jnp.concatenate([fpos[:, None] * xw_h, fpos[:, None], pad], axis=1)
    Tn = jnp.concatenate([fneg[:, None] * xw_h, fneg[:, None], pad], axis=1)
    Sp = _edge_pass(Tp + token, als, ald, 0.0, src3, dst3)
    token = 0.0 * Sp[0, 0]
    Sn = _edge_pass(Tn + token, als, ald, 1.0, src3, dst3)
    token = 0.0 * Sn[0, 0]
    gpos = jnp.exp(ald - adm + bb - mb)
    gneg = jnp.exp(0.2 * (ald - adm) + 0.2 * bb - mb)
    num = gpos[:, None] * Sp[:, :HID] + gneg[:, None] * Sn[:, :HID]
    den = gpos * Sp[:, HID] + gneg * Sn[:, HID]
    return num, den, token


def kernel(x, edge_index, n2v_table, W_n2vp, b_n2vp, W_in, b_in, W_gate,
           b_gate, Wl1, bl1, Wr1, Wl2, bl2, Wr2, W_gat, att_src, att_dst,
           b_gat, Wl3, bl3, Wr3):
    n = x.shape[0]
    src3 = edge_index[0].astype(jnp.int32).reshape(NW, NCHUNK, CH)
    dst3 = edge_index[1].astype(jnp.int32).reshape(NW, NCHUNK, CH)
    h0 = _h0(x, n2v_table, W_n2vp, b_n2vp, W_in, b_in, W_gate, b_gate)
    h0e = jnp.concatenate(
        [h0, jnp.ones((n, 1), jnp.float32), jnp.zeros((n, 15), jnp.float32)],
        axis=1)
    s1 = _segsum(h0e, src3, dst3)
    deg_inv = 1.0 / jnp.maximum(s1[:, HID], 1.0)
    h1 = jax.nn.relu(s1[:, :HID] * deg_inv[:, None] @ Wl1.T + bl1 + h0 @ Wr1.T)
    h1e = jnp.concatenate([h1, jnp.zeros((n, 16), jnp.float32)], axis=1)
    s2 = _segsum(h1e + 0.0 * s1[0, 0], src3, dst3)
    h2 = jax.nn.relu(s2[:, :HID] * deg_inv[:, None] @ Wl2.T + bl2 + h1 @ Wr2.T)

    xw = h2 @ W_gat.T
    outs = []
    token = 0.0 * s2[0, 0]
    for h in range(HEADS):
        xw_h = xw[:, h * HID:(h + 1) * HID]
        als = xw_h @ att_src[h]
        ald = xw_h @ att_dst[h]
        num, den, token = _gat_head(xw_h, als, ald, src3, dst3, token)
        outs.append(num / jnp.where(den > 0, den, 1.0)[:, None])
    h3 = jax.nn.relu((outs[0] + outs[1]) * 0.5 + b_gat)

    h3e = jnp.concatenate([h3, jnp.zeros((n, 16), jnp.float32)], axis=1)
    s3 = _segsum(h3e + token, src3, dst3)
    return s3[:, :HID] * deg_inv[:, None] @ Wl3.T + bl3 + h3 @ Wr3.T


# speedup vs baseline: 12.5381x; 12.5381x over previous
import dataclasses
import functools

import jax
import jax.numpy as jnp
from jax import lax
from jax.experimental import pallas as pl
from jax.experimental.pallas import tpu as pltpu
from jax.experimental.pallas import tpu_sc as plsc

N = 10000
E = 320000
D_IN = 128
HID = 64
OUT = 32
EMB = 128
HEADS = 2

NC = 2
NS = 16
NW = NC * NS
EPW = E // NW
CH = 80
NCHUNK = EPW // CH
LL = 16


def _h0_body(x_ref, n2v_ref, Wn_ref, bn_ref, Win_ref, bin_ref, Wg_ref, bg_ref,
             o_ref):
    x = x_ref[...]
    emb_p = n2v_ref[...] @ Wn_ref[...].T + bn_ref[...]
    Win = Win_ref[...]
    raw = x @ Win[:, :D_IN].T + bin_ref[...]
    xp = raw + emb_p @ Win[:, D_IN:].T
    Wg = Wg_ref[...]
    g = jax.nn.sigmoid(x @ Wg[:, :D_IN].T + emb_p @ Wg[:, D_IN:].T + bg_ref[...])
    o_ref[...] = g * xp + (1.0 - g) * raw


def _h0(x, n2v_table, W_n2vp, b_n2vp, W_in, b_in, W_gate, b_gate):
    blk = 2000
    grid = (N // blk,)
    row_spec = pl.BlockSpec((blk, D_IN), lambda i: (i, 0))
    full = lambda shape: pl.BlockSpec(shape, lambda i: (0,) * len(shape))
    return pl.pallas_call(
        _h0_body,
        grid=grid,
        in_specs=[row_spec, row_spec, full((HID, EMB)), full((1, HID)),
                  full((HID, D_IN + HID)), full((1, HID)),
                  full((HID, D_IN + HID)), full((1, HID))],
        out_specs=pl.BlockSpec((blk, HID), lambda i: (i, 0)),
        out_shape=jax.ShapeDtypeStruct((N, HID), jnp.float32),
    )(x, n2v_table, W_n2vp, b_n2vp.reshape(1, HID), W_in,
      b_in.reshape(1, HID), W_gate, b_gate.reshape(1, HID))


def _stripe_copy(src_ref, dst_ref, sid, nrows):
    rp = (nrows // NS) & ~7
    tail = nrows - NS * rp
    pltpu.sync_copy(src_ref.at[pl.ds(sid * rp, rp)],
                    dst_ref.at[pl.ds(sid * rp, rp)])
    if tail:
        @pl.when(sid == 0)
        def _():
            pltpu.sync_copy(src_ref.at[pl.ds(NS * rp, tail)],
                            dst_ref.at[pl.ds(NS * rp, tail)])


def _edge_ring(h_hbm, gidx2, sidx2, acc_sh, rows_a, rows_b, sem_a, sem_b):
    pltpu.async_copy(h_hbm.at[gidx2.at[0]], rows_a, sem_a)

    @pl.loop(0, NCHUNK - 1, step=2)
    def _(i):
        pltpu.make_async_copy(h_hbm.at[gidx2.at[i]], rows_a, sem_a).wait()
        pltpu.async_copy(h_hbm.at[gidx2.at[i + 1]], rows_b, sem_b)
        pltpu.sync_copy(rows_a, acc_sh.at[sidx2.at[i]], add=True)
        pltpu.make_async_copy(h_hbm.at[gidx2.at[i + 1]], rows_b, sem_b).wait()
        pltpu.async_copy(h_hbm.at[gidx2.at[i + 2]], rows_a, sem_a)
        pltpu.sync_copy(rows_b, acc_sh.at[sidx2.at[i + 1]], add=True)

    last = NCHUNK - 1
    pltpu.make_async_copy(h_hbm.at[gidx2.at[last]], rows_a, sem_a).wait()
    pltpu.sync_copy(rows_a, acc_sh.at[sidx2.at[last]], add=True)


D80 = 80
MP = N + 512
_MESH = plsc.VectorSubcoreMesh(core_axis_name="c", subcore_axis_name="s")
_CP = pltpu.CompilerParams(use_tc_tiling_on_sc=False)
if "needs_layout_passes" in pltpu.CompilerParams.__dataclass_fields__:
    _CP = dataclasses.replace(_CP, needs_layout_passes=False)


def _edge_kernel_body(t_hbm, als_hbm, ald_hbm, flag_hbm, src_hbm, dst_hbm,
                      z_hbm, out_hbm, src2_v, dst2_v, sidx2_v,
                      rows_a, rows_b, als_v, ald_v, flag_v, acc_sh, sem_a,
                      sem_b):
    cid = lax.axis_index("c")
    sid = lax.axis_index("s")
    wid = sid * NC + cid
    pltpu.sync_copy(src_hbm.at[wid], src2_v)
    pltpu.sync_copy(dst_hbm.at[wid], dst2_v)
    pltpu.sync_copy(als_hbm, als_v)
    pltpu.sync_copy(ald_hbm, ald_v)
    pltpu.sync_copy(flag_hbm, flag_v)
    _stripe_copy(z_hbm, acc_sh, sid, MP)

    flip = flag_v[...] > 0.5
    lane = lax.iota(jnp.int32, LL)

    @pl.loop(0, NCHUNK)
    def _(i):
        for kk in range(CH // LL):
            s16 = src2_v[i, pl.ds(kk * LL, LL)]
            d16 = dst2_v[i, pl.ds(kk * LL, LL)]
            a = plsc.load_gather(als_v, [s16])
            b = plsc.load_gather(ald_v, [d16])
            keep = jnp.logical_xor((a + b) > 0.0, flip)
            dump = (N + ((i * 5 + kk) % 32) * LL) + lane
            sidx2_v[i, pl.ds(kk * LL, LL)] = jnp.where(keep, d16, dump)

    plsc.subcore_barrier()
    _edge_ring(t_hbm, src2_v, sidx2_v, acc_sh, rows_a, rows_b, sem_a, sem_b)
    plsc.subcore_barrier()
    _stripe_copy(acc_sh, out_hbm.at[cid], sid, MP)


_EDGE_KERNEL = pl.kernel(
    _edge_kernel_body,
    out_type=jax.ShapeDtypeStruct((NC, MP, D80), jnp.float32),
    mesh=_MESH,
    compiler_params=_CP,
    scratch_types=[
        pltpu.VMEM((NCHUNK, CH), jnp.int32),
        pltpu.VMEM((NCHUNK, CH), jnp.int32),
        pltpu.VMEM((NCHUNK, CH), jnp.int32),
        pltpu.VMEM((CH, D80), jnp.float32),
        pltpu.VMEM((CH, D80), jnp.float32),
        pltpu.VMEM((N,), jnp.float32),
        pltpu.VMEM((N,), jnp.float32),
        pltpu.VMEM((LL,), jnp.float32),
        pltpu.VMEM_SHARED((MP, D80), jnp.float32),
        pltpu.SemaphoreType.DMA,
        pltpu.SemaphoreType.DMA,
    ],
)

_DUMP = None


def _edge_pass(T, als, ald, flag, src3, dst3):
    zeros = jnp.zeros((MP, D80), jnp.float32)
    flagv = jnp.full((LL,), flag, jnp.float32)
    p = _EDGE_KERNEL(T, als, ald, flagv, src3, dst3, zeros)
    return p[0, :N] + p[1, :N]


_ONES = None


def _segsum(h_ext, src3, dst3):
    ones = jnp.ones((N,), jnp.float32)
    return _edge_pass(h_ext, ones, ones, 0.0, src3, dst3)


def _gat_head(xw_h, als, ald, src3, dst3, token):
    n = xw_h.shape[0]
    asm = jnp.max(als)
    adm = jnp.max(ald)
    bb = asm + adm
    mb = jnp.maximum(bb, 0.2 * bb)
    fpos = jnp.exp(als - asm)
    fneg = jnp.exp(0.2 * (als - asm))
    pad = jnp.zeros((n, 15), jnp.float32)
    Tp = jnp.concatenate([fpos[:, None] * xw_h, fpos[:, None], pad], axis=1)
    Tn = jnp.concatenate([fneg[:, None] * xw_h, fneg[:, None], pad], axis=1)
    Sp = _edge_pass(Tp + token, als, ald, 0.0, src3, dst3)
    token = 0.0 * Sp[0, 0]
    Sn = _edge_pass(Tn + token, als, ald, 1.0, src3, dst3)
    token = 0.0 * Sn[0, 0]
    gpos = jnp.exp(ald - adm + bb - mb)
    gneg = jnp.exp(0.2 * (ald - adm) + 0.2 * bb - mb)
    num = gpos[:, None] * Sp[:, :HID] + gneg[:, None] * Sn[:, :HID]
    den = gpos * Sp[:, HID] + gneg * Sn[:, HID]
    return num, den, token


def kernel(x, edge_index, n2v_table, W_n2vp, b_n2vp, W_in, b_in, W_gate,
           b_gate, Wl1, bl1, Wr1, Wl2, bl2, Wr2, W_gat, att_src, att_dst,
           b_gat, Wl3, bl3, Wr3):
    n = x.shape[0]
    src3 = edge_index[0].astype(jnp.int32).reshape(NW, NCHUNK, CH)
    dst3 = edge_index[1].astype(jnp.int32).reshape(NW, NCHUNK, CH)
    h0 = _h0(x, n2v_table, W_n2vp, b_n2vp, W_in, b_in, W_gate, b_gate)
    h0e = jnp.concatenate(
        [h0, jnp.ones((n, 1), jnp.float32), jnp.zeros((n, 15), jnp.float32)],
        axis=1)
    s1 = _segsum(h0e, src3, dst3)
    deg_inv = 1.0 / jnp.maximum(s1[:, HID], 1.0)
    h1 = jax.nn.relu(s1[:, :HID] * deg_inv[:, None] @ Wl1.T + bl1 + h0 @ Wr1.T)
    h1e = jnp.concatenate([h1, jnp.zeros((n, 16), jnp.float32)], axis=1)
    s2 = _segsum(h1e + 0.0 * s1[0, 0], src3, dst3)
    h2 = jax.nn.relu(s2[:, :HID] * deg_inv[:, None] @ Wl2.T + bl2 + h1 @ Wr2.T)

    xw = h2 @ W_gat.T
    outs = []
    token = 0.0 * s2[0, 0]
    for h in range(HEADS):
        xw_h = xw[:, h * HID:(h + 1) * HID]
        als = xw_h @ att_src[h]
        ald = xw_h @ att_dst[h]
        num, den, token = _gat_head(xw_h, als, ald, src3, dst3, token)
        outs.append(num / jnp.where(den > 0, den, 1.0)[:, None])
    h3 = jax.nn.relu((outs[0] + outs[1]) * 0.5 + b_gat)

    h3e = jnp.concatenate([h3, jnp.zeros((n, 16), jnp.float32)], axis=1)
    s3 = _segsum(h3e + token, src3, dst3)
    return s3[:, :HID] * deg_inv[:, None] @ Wl3.T + bl3 + h3 @ Wr3.T
